# baseline (speedup 1.0000x reference)
import jax
import jax.numpy as jnp
from jax.experimental import pallas as pl
from jax.experimental.pallas import tpu as pltpu

_BEG_K = 5
_END_K = 5
_EPS = 1e-12
_NEG = -1e30


def _squad_head_kernel(cls_ref, hid_ref, pmT_ref, Wst_ref, bst_ref,
                       We0a_ref, We0b_ref, be0_ref, lng_ref, lnb_ref,
                       We1_ref, be1_ref, Wa0a_ref, Wa0b_ref, ba0_ref,
                       Wa1_ref,
                       tbv_ref, tbi_ref, tev_ref, tei_ref, cls_out_ref):
    S, H = hid_ref.shape[1], hid_ref.shape[2]
    hid = hid_ref[0]
    pm = pmT_ref[0]

    lb = jnp.dot(hid, Wst_ref[...], preferred_element_type=jnp.float32)
    lb = (lb + bst_ref[...]) * (1.0 - pm) + _NEG * pm
    m = jnp.max(lb, axis=0, keepdims=True)
    e = jnp.exp(lb - m)
    d = jnp.sum(e, axis=0, keepdims=True)

    lbT = lb.reshape(1, S)
    iotaL = jax.lax.broadcasted_iota(jnp.int32, (1, S), 1)
    vals, idxs, rows = [], [], []
    for _ in range(_BEG_K):
        mv = jnp.max(lbT, axis=1, keepdims=True)
        mi = jnp.min(jnp.where(lbT == mv, iotaL, S), axis=1,
                     keepdims=True)
        lbT = jnp.where(iotaL == mi, -jnp.inf, lbT)
        rows.append(hid_ref[0, pl.dslice(mi[0, 0], 1), :])
        vals.append(mv)
        idxs.append(mi)
    tbv_ref[...] = (jnp.exp(jnp.concatenate(vals, axis=1) - m) / d
                    ).reshape(1, 1, _BEG_K)
    tbi_ref[...] = jnp.concatenate(idxs, axis=1).reshape(1, 1, _BEG_K)
    x_beg = jnp.concatenate(rows, axis=0)

    hidproj = jnp.dot(hid, We0a_ref[...], preferred_element_type=jnp.float32)
    xproj = jnp.dot(x_beg, We0b_ref[...], preferred_element_type=jnp.float32)
    xproj = xproj + be0_ref[...]
    lng = lng_ref[...]
    lnb = lnb_ref[...]
    le_cols = []
    ones_col = jnp.full((H, 1), 1.0, jnp.float32)
    inv_h = jnp.float32(1.0 / H)
    for k in range(_BEG_K):
        hk = jnp.tanh(hidproj + xproj[k:k + 1, :])
        mu = jnp.dot(hk, ones_col,
                     preferred_element_type=jnp.float32) * inv_h
        xc = hk - mu
        var = jnp.mean(xc * xc, axis=1, keepdims=True)
        hn = xc * jax.lax.rsqrt(var + _EPS) * lng + lnb
        le_cols.append(jnp.dot(hn, We1_ref[...],
                               preferred_element_type=jnp.float32))
    le = jnp.concatenate(le_cols, axis=1) + be1_ref[...]
    le = le * (1.0 - pm) + _NEG * pm

    leT = le.T
    me = jnp.max(leT, axis=1, keepdims=True)
    eeT = jnp.exp(leT - me)
    denom = jnp.sum(eeT, axis=1, keepdims=True)
    iotaE = jax.lax.broadcasted_iota(jnp.int32, (_BEG_K, S), 1)
    ev, ei = [], []
    for _ in range(_END_K):
        mv = jnp.max(eeT, axis=1, keepdims=True)
        mi = jnp.min(jnp.where(eeT == mv, iotaE, S), axis=1,
                     keepdims=True)
        eeT = jnp.where(iotaE == mi, -1.0, eeT)
        ev.append(mv / denom)
        ei.append(mi)
    tev_ref[...] = jnp.concatenate(ev, axis=1).reshape(1, _BEG_K, _END_K)
    tei_ref[...] = jnp.concatenate(ei, axis=1).reshape(1, _BEG_K, _END_K)

    p_beg = e / d
    xcls = jax.lax.dot_general(p_beg, hid, (((0,), (0,)), ((), ())),
                               preferred_element_type=jnp.float32)
    ci = cls_ref[pl.program_id(0)]
    ctok = hid_ref[0, pl.dslice(ci, 1), :]
    h2 = jnp.tanh(jnp.dot(xcls, Wa0a_ref[...],
                          preferred_element_type=jnp.float32)
                  + jnp.dot(ctok, Wa0b_ref[...],
                            preferred_element_type=jnp.float32)
                  + ba0_ref[...])
    cls_out_ref[...] = jnp.dot(h2, Wa1_ref[...],
                               preferred_element_type=jnp.float32
                               ).reshape(1, 1, 1)


def kernel(hiddens, cls_index, p_mask, W_start, b_start, W_e0, b_e0,
           ln_g, ln_b, W_e1, b_e1, W_a0, b_a0, W_a1):
    B, S, H = hiddens.shape
    f32 = jnp.float32
    pmT = p_mask.reshape(B, S, 1).astype(f32)
    cls_i = cls_index.astype(jnp.int32)

    args = (
        cls_i,
        hiddens,
        pmT,
        W_start,
        b_start.reshape(1, 1),
        W_e0[:H, :], W_e0[H:, :],
        b_e0.reshape(1, H),
        ln_g.reshape(1, H), ln_b.reshape(1, H),
        W_e1,
        b_e1.reshape(1, 1),
        W_a0[:H, :], W_a0[H:, :],
        b_a0.reshape(1, H),
        W_a1,
    )
    const = lambda *shape: pl.BlockSpec(shape, lambda b: (0,) * len(shape))
    in_specs = [
        pl.BlockSpec(memory_space=pltpu.SMEM),
        pl.BlockSpec((1, S, H), lambda b: (b, 0, 0)),
        pl.BlockSpec((1, S, 1), lambda b: (b, 0, 0)),
        const(H, 1), const(1, 1),
        const(H, H), const(H, H), const(1, H),
        const(1, H), const(1, H),
        const(H, 1), const(1, 1),
        const(H, H), const(H, H), const(1, H),
        const(H, 1),
    ]
    out_specs = [
        pl.BlockSpec((1, 1, _BEG_K), lambda b: (b, 0, 0)),
        pl.BlockSpec((1, 1, _BEG_K), lambda b: (b, 0, 0)),
        pl.BlockSpec((1, _BEG_K, _END_K), lambda b: (b, 0, 0)),
        pl.BlockSpec((1, _BEG_K, _END_K), lambda b: (b, 0, 0)),
        pl.BlockSpec((1, 1, 1), lambda b: (b, 0, 0)),
    ]
    out_shape = [
        jax.ShapeDtypeStruct((B, 1, _BEG_K), f32),
        jax.ShapeDtypeStruct((B, 1, _BEG_K), jnp.int32),
        jax.ShapeDtypeStruct((B, _BEG_K, _END_K), f32),
        jax.ShapeDtypeStruct((B, _BEG_K, _END_K), jnp.int32),
        jax.ShapeDtypeStruct((B, 1, 1), f32),
    ]
    tbv, tbi, tev, tei, cls_out = pl.pallas_call(
        _squad_head_kernel,
        grid=(B,),
        in_specs=in_specs,
        out_specs=out_specs,
        out_shape=out_shape,
    )(*args)
    return (tbv.reshape(B, _BEG_K), tbi.reshape(B, _BEG_K),
            jnp.swapaxes(tev, 1, 2).reshape(B, _END_K * _BEG_K),
            jnp.swapaxes(tei, 1, 2).reshape(B, _END_K * _BEG_K),
            cls_out.reshape(B))


# speedup vs baseline: 1.1810x; 1.1810x over previous
import jax
import jax.numpy as jnp
from jax.experimental import pallas as pl
from jax.experimental.pallas import tpu as pltpu

_BEG_K = 5
_END_K = 5
_EPS = 1e-12
_NEG = -1e30


def _squad_head_kernel(cls_ref, hid_ref, pmT_ref, Wst_ref, bst_ref,
                       We0a_ref, We0b_ref, be0_ref, lng_ref, lnb_ref,
                       We1_ref, be1_ref, Wa0a_ref, Wa0b_ref, ba0_ref,
                       Wa1_ref,
                       tbv_ref, tbi_ref, tev_ref, tei_ref, cls_out_ref):
    S, H = hid_ref.shape[1], hid_ref.shape[2]
    hid = hid_ref[0]
    pm = pmT_ref[0]

    lb = jnp.dot(hid, Wst_ref[...], preferred_element_type=jnp.float32)
    lb = (lb + bst_ref[...]) * (1.0 - pm) + _NEG * pm
    m = jnp.max(lb, axis=0, keepdims=True)
    e = jnp.exp(lb - m)
    d = jnp.sum(e, axis=0, keepdims=True)

    lbT = lb.reshape(1, S)
    iotaL = jax.lax.broadcasted_iota(jnp.int32, (1, S), 1)
    vals, idxs, rows = [], [], []
    for _ in range(_BEG_K):
        mv = jnp.max(lbT, axis=1, keepdims=True)
        mi = jnp.min(jnp.where(lbT == mv, iotaL, S), axis=1,
                     keepdims=True)
        lbT = jnp.where(iotaL == mi, -jnp.inf, lbT)
        rows.append(hid_ref[0, pl.dslice(mi[0, 0], 1), :])
        vals.append(mv)
        idxs.append(mi)
    tbv_ref[...] = (jnp.exp(jnp.concatenate(vals, axis=1) - m) / d
                    ).reshape(1, 1, _BEG_K)
    tbi_ref[...] = jnp.concatenate(idxs, axis=1).reshape(1, 1, _BEG_K)
    x_beg = jnp.concatenate(rows, axis=0)

    hidproj = jnp.dot(hid, We0a_ref[...], preferred_element_type=jnp.float32)
    xproj = jnp.dot(x_beg, We0b_ref[...], preferred_element_type=jnp.float32)
    xproj = xproj + be0_ref[...]
    lng = lng_ref[...]
    lnb = lnb_ref[...]
    le_cols = []
    for k in range(_BEG_K):
        hk = jnp.tanh(hidproj + xproj[k:k + 1, :])
        mu = jnp.mean(hk, axis=1, keepdims=True)
        xc = hk - mu
        var = jnp.mean(xc * xc, axis=1, keepdims=True)
        hn = xc * jax.lax.rsqrt(var + _EPS) * lng + lnb
        le_cols.append(jnp.dot(hn, We1_ref[...],
                               preferred_element_type=jnp.float32))
    le = jnp.concatenate(le_cols, axis=1) + be1_ref[...]
    le = le * (1.0 - pm) + _NEG * pm

    leT = le.T
    me = jnp.max(leT, axis=1, keepdims=True)
    eeT = jnp.exp(leT - me)
    denom = jnp.sum(eeT, axis=1, keepdims=True)
    iotaE = jax.lax.broadcasted_iota(jnp.int32, (_BEG_K, S), 1)
    ev, ei = [], []
    for _ in range(_END_K):
        mv = jnp.max(eeT, axis=1, keepdims=True)
        mi = jnp.min(jnp.where(eeT == mv, iotaE, S), axis=1,
                     keepdims=True)
        eeT = jnp.where(iotaE == mi, -1.0, eeT)
        ev.append(mv / denom)
        ei.append(mi)
    tev_ref[...] = jnp.concatenate(ev, axis=1).reshape(1, _BEG_K, _END_K)
    tei_ref[...] = jnp.concatenate(ei, axis=1).reshape(1, _BEG_K, _END_K)

    p_beg = e / d
    xcls = jax.lax.dot_general(p_beg, hid, (((0,), (0,)), ((), ())),
                               preferred_element_type=jnp.float32)
    ci = cls_ref[pl.program_id(0)]
    ctok = hid_ref[0, pl.dslice(ci, 1), :]
    h2 = jnp.tanh(jnp.dot(xcls, Wa0a_ref[...],
                          preferred_element_type=jnp.float32)
                  + jnp.dot(ctok, Wa0b_ref[...],
                            preferred_element_type=jnp.float32)
                  + ba0_ref[...])
    cls_out_ref[...] = jnp.dot(h2, Wa1_ref[...],
                               preferred_element_type=jnp.float32
                               ).reshape(1, 1, 1)


def kernel(hiddens, cls_index, p_mask, W_start, b_start, W_e0, b_e0,
           ln_g, ln_b, W_e1, b_e1, W_a0, b_a0, W_a1):
    B, S, H = hiddens.shape
    f32 = jnp.float32
    pmT = p_mask.reshape(B, S, 1).astype(f32)
    cls_i = cls_index.astype(jnp.int32)

    args = (
        cls_i,
        hiddens,
        pmT,
        W_start,
        b_start.reshape(1, 1),
        W_e0[:H, :], W_e0[H:, :],
        b_e0.reshape(1, H),
        ln_g.reshape(1, H), ln_b.reshape(1, H),
        W_e1,
        b_e1.reshape(1, 1),
        W_a0[:H, :], W_a0[H:, :],
        b_a0.reshape(1, H),
        W_a1,
    )
    const = lambda *shape: pl.BlockSpec(shape, lambda b: (0,) * len(shape))
    in_specs = [
        pl.BlockSpec(memory_space=pltpu.SMEM),
        pl.BlockSpec((1, S, H), lambda b: (b, 0, 0)),
        pl.BlockSpec((1, S, 1), lambda b: (b, 0, 0)),
        const(H, 1), const(1, 1),
        const(H, H), const(H, H), const(1, H),
        const(1, H), const(1, H),
        const(H, 1), const(1, 1),
        const(H, H), const(H, H), const(1, H),
        const(H, 1),
    ]
    out_specs = [
        pl.BlockSpec((1, 1, _BEG_K), lambda b: (b, 0, 0)),
        pl.BlockSpec((1, 1, _BEG_K), lambda b: (b, 0, 0)),
        pl.BlockSpec((1, _BEG_K, _END_K), lambda b: (b, 0, 0)),
        pl.BlockSpec((1, _BEG_K, _END_K), lambda b: (b, 0, 0)),
        pl.BlockSpec((1, 1, 1), lambda b: (b, 0, 0)),
    ]
    out_shape = [
        jax.ShapeDtypeStruct((B, 1, _BEG_K), f32),
        jax.ShapeDtypeStruct((B, 1, _BEG_K), jnp.int32),
        jax.ShapeDtypeStruct((B, _BEG_K, _END_K), f32),
        jax.ShapeDtypeStruct((B, _BEG_K, _END_K), jnp.int32),
        jax.ShapeDtypeStruct((B, 1, 1), f32),
    ]
    tbv, tbi, tev, tei, cls_out = pl.pallas_call(
        _squad_head_kernel,
        grid=(B,),
        in_specs=in_specs,
        out_specs=out_specs,
        out_shape=out_shape,
    )(*args)
    return (tbv.reshape(B, _BEG_K), tbi.reshape(B, _BEG_K),
            jnp.swapaxes(tev, 1, 2).reshape(B, _END_K * _BEG_K),
            jnp.swapaxes(tei, 1, 2).reshape(B, _END_K * _BEG_K),
            cls_out.reshape(B))
